# baseline (speedup 1.0000x reference)
import jax
import jax.numpy as jnp
from jax.experimental import pallas as pl
from jax.experimental.pallas import tpu as pltpu

_NE = 16
_TOPK = 2
_AUXW = 0.01


def _gate_kernel(x_ref, w_ref, out_ref, aux_ref, acc_ref, *, n_rows):
    i = pl.program_id(0)
    nb = pl.num_programs(0)

    x = x_ref[...]
    w = w_ref[...]
    logits = jnp.dot(x, w, preferred_element_type=jnp.float32)

    m = jnp.max(logits, axis=-1, keepdims=True)
    e = jnp.exp(logits - m)
    s = jnp.dot(e, jnp.ones((_NE, _NE), jnp.float32),
                preferred_element_type=jnp.float32)
    gate = e / s
    out_ref[...] = gate

    l2 = jnp.where(logits == m, -jnp.inf, logits)
    m2 = jnp.max(l2, axis=-1, keepdims=True)
    mask = (logits >= m2).astype(jnp.float32)

    gsum = jnp.sum(gate, axis=0, keepdims=True)
    csum = jnp.sum(mask, axis=0, keepdims=True)
    part = jnp.concatenate([gsum, csum], axis=0)

    @pl.when(i == 0)
    def _():
        acc_ref[...] = part

    @pl.when(i > 0)
    def _():
        acc_ref[...] = acc_ref[...] + part

    @pl.when(i == nb - 1)
    def _():
        avg = acc_ref[0:1, :] * (1.0 / n_rows)
        load = acc_ref[1:2, :] * (1.0 / (_TOPK * n_rows))
        aux_ref[0, 0] = _AUXW * _NE * jnp.sum(avg * load)


def kernel(x, expert_queries, temperature):
    B, H, W, dim = x.shape
    n = B * H * W
    x_flat = x.reshape(n, dim)
    wt = (expert_queries / temperature).T

    rows = 4096
    grid = n // rows

    import functools
    gate_flat, aux = pl.pallas_call(
        functools.partial(_gate_kernel, n_rows=n),
        grid=(grid,),
        in_specs=[
            pl.BlockSpec((rows, dim), lambda i: (i, 0)),
            pl.BlockSpec((dim, _NE), lambda i: (0, 0)),
        ],
        out_specs=[
            pl.BlockSpec((rows, _NE), lambda i: (i, 0)),
            pl.BlockSpec(memory_space=pltpu.SMEM),
        ],
        out_shape=[
            jax.ShapeDtypeStruct((n, _NE), jnp.float32),
            jax.ShapeDtypeStruct((1, 1), jnp.float32),
        ],
        scratch_shapes=[pltpu.VMEM((2, _NE), jnp.float32)],
    )(x_flat, wt)

    return gate_flat.reshape(B, H, W, _NE), aux[0, 0]


# speedup vs baseline: 1.0822x; 1.0822x over previous
import jax
import jax.numpy as jnp
from jax.experimental import pallas as pl
from jax.experimental.pallas import tpu as pltpu

_NE = 16
_TOPK = 2
_AUXW = 0.01


def _gate_kernel(x_ref, w_ref, out_ref, aux_ref, acc_ref, *, n_rows):
    i = pl.program_id(0)
    nb = pl.num_programs(0)

    x = x_ref[...]
    w = w_ref[...]
    logits = jnp.dot(x, w, preferred_element_type=jnp.float32)

    m = jnp.max(logits, axis=-1, keepdims=True)
    e = jnp.exp(logits - m)
    s = jnp.dot(e, jnp.ones((_NE, _NE), jnp.float32),
                preferred_element_type=jnp.float32)
    gate = e / s
    out_ref[...] = gate

    l2 = jnp.where(logits == m, -jnp.inf, logits)
    m2 = jnp.max(l2, axis=-1, keepdims=True)
    mask = (logits >= m2).astype(jnp.float32)

    gsum = jnp.sum(gate, axis=0, keepdims=True)
    csum = jnp.sum(mask, axis=0, keepdims=True)
    part = jnp.concatenate([gsum, csum], axis=0)

    @pl.when(i == 0)
    def _():
        acc_ref[...] = part

    @pl.when(i > 0)
    def _():
        acc_ref[...] = acc_ref[...] + part

    @pl.when(i == nb - 1)
    def _():
        avg = acc_ref[0:1, :] * (1.0 / n_rows)
        load = acc_ref[1:2, :] * (1.0 / (_TOPK * n_rows))
        aux_ref[0, 0] = _AUXW * _NE * jnp.sum(avg * load)


def kernel(x, expert_queries, temperature):
    B, H, W, dim = x.shape
    n = B * H * W
    x_flat = x.reshape(n, dim)
    wt = (expert_queries / temperature).T

    rows = 16384
    grid = n // rows

    import functools
    gate_flat, aux = pl.pallas_call(
        functools.partial(_gate_kernel, n_rows=n),
        grid=(grid,),
        in_specs=[
            pl.BlockSpec((rows, dim), lambda i: (i, 0)),
            pl.BlockSpec((dim, _NE), lambda i: (0, 0)),
        ],
        out_specs=[
            pl.BlockSpec((rows, _NE), lambda i: (i, 0)),
            pl.BlockSpec(memory_space=pltpu.SMEM),
        ],
        out_shape=[
            jax.ShapeDtypeStruct((n, _NE), jnp.float32),
            jax.ShapeDtypeStruct((1, 1), jnp.float32),
        ],
        scratch_shapes=[pltpu.VMEM((2, _NE), jnp.float32)],
    )(x_flat, wt)

    return gate_flat.reshape(B, H, W, _NE), aux[0, 0]
